# baseline (speedup 1.0000x reference)
import functools

import jax
import jax.numpy as jnp
import numpy as np
from jax import lax
from jax.experimental import pallas as pl
from jax.experimental.pallas import tpu as pltpu
from jax.experimental.pallas import tpu_sc as plsc

_B = 2
_Z = 3
_G = 33
_F = 3
_R = (_G - 1) * _F + 1
_M = _R * _R
_N = 4096
_LANES = 16
_NC = 2
_NS = 16
_NW = _NC * _NS
_QPW = _N // _NS
_MPAD = 9472
_PADVAL = 1e8

_QV = 4
_PU = 8


def _interp_matrix() -> np.ndarray:
    w = np.zeros((_G, _R), dtype=np.float32)
    for p in range(_R):
        j = p // _F
        k = np.float32(p % _F) / np.float32(_F)
        jn = min(j + 1, _G - 1)
        w[j, p] += np.float32(1.0) - k
        w[jn, p] += k
    return w


def _refine_body(wt_ref, w_ref, m_ref, o_ref):
    m = m_ref[0]
    t = lax.dot_general(
        wt_ref[...], m, (((1,), (0,)), ((), ())),
        precision=lax.Precision.HIGHEST,
        preferred_element_type=jnp.float32)
    o_ref[0] = lax.dot_general(
        t, w_ref[...], (((1,), (0,)), ((), ())),
        precision=lax.Precision.HIGHEST,
        preferred_element_type=jnp.float32)


def _refine(network_mesh):
    mesh6 = network_mesh.reshape(_B * _Z, _G, _G)
    w = jnp.asarray(_interp_matrix())
    wt = w.T
    return pl.pallas_call(
        _refine_body,
        grid=(_B * _Z,),
        in_specs=[
            pl.BlockSpec((_R, _G), lambda i: (0, 0)),
            pl.BlockSpec((_G, _R), lambda i: (0, 0)),
            pl.BlockSpec((1, _G, _G), lambda i: (i, 0, 0)),
        ],
        out_specs=pl.BlockSpec((1, _R, _R), lambda i: (i, 0, 0)),
        out_shape=jax.ShapeDtypeStruct((_B * _Z, _R, _R), jnp.float32),
    )(wt, w, mesh6)


def _round_bf16(v):
    u = plsc.bitcast(v, jnp.uint32)
    lsb = jnp.bitwise_and(jnp.right_shift(u, jnp.uint32(16)), jnp.uint32(1))
    r = jnp.bitwise_and(u + jnp.uint32(0x7FFF) + lsb, jnp.uint32(0xFFFF0000))
    return plsc.bitcast(r, jnp.float32)


def _chamfer_body(meshf_hbm, pc_hbm, out_hbm,
                  mx_v, my_v, mz_v, aa_v, px_v, py_v, pz_v, sum_v):
    c = lax.axis_index("c")
    s = lax.axis_index("s")
    b = c
    w = s * _NC + c
    qoff = s * _QPW

    pltpu.sync_copy(meshf_hbm.at[pl.ds((b * _Z + 0) * _MPAD, _MPAD)], mx_v)
    pltpu.sync_copy(meshf_hbm.at[pl.ds((b * _Z + 1) * _MPAD, _MPAD)], my_v)
    pltpu.sync_copy(meshf_hbm.at[pl.ds((b * _Z + 2) * _MPAD, _MPAD)], mz_v)
    pltpu.sync_copy(pc_hbm.at[pl.ds((b * _Z + 0) * _N + qoff, _QPW)], px_v)
    pltpu.sync_copy(pc_hbm.at[pl.ds((b * _Z + 1) * _N + qoff, _QPW)], py_v)
    pltpu.sync_copy(pc_hbm.at[pl.ds((b * _Z + 2) * _N + qoff, _QPW)], pz_v)

    def prep(vi, _):
        sl = pl.ds(vi * _LANES, _LANES)
        a = mx_v[sl]
        b2 = my_v[sl]
        cz = mz_v[sl]
        aa_v[sl] = a * a + b2 * b2 + cz * cz
        mx_v[sl] = _round_bf16(a)
        my_v[sl] = _round_bf16(b2)
        mz_v[sl] = _round_bf16(cz)
        return 0
    lax.fori_loop(0, _MPAD // _LANES, prep, 0, unroll=1)

    total = jnp.zeros((_LANES,), jnp.float32)
    n_groups = _QPW // (_QV * _LANES)
    for g in range(n_groups):
        qxr = [px_v[pl.ds((g * _QV + i) * _LANES, _LANES)] for i in range(_QV)]
        qyr = [py_v[pl.ds((g * _QV + i) * _LANES, _LANES)] for i in range(_QV)]
        qzr = [pz_v[pl.ds((g * _QV + i) * _LANES, _LANES)] for i in range(_QV)]
        bb = [qxr[i] * qxr[i] + qyr[i] * qyr[i] + qzr[i] * qzr[i]
              for i in range(_QV)]
        qx = [_round_bf16(qxr[i]) for i in range(_QV)]
        qy = [_round_bf16(qyr[i]) for i in range(_QV)]
        qz = [_round_bf16(qzr[i]) for i in range(_QV)]

        def mbody(mi, accs, qx=qx, qy=qy, qz=qz, bb=bb):
            accs = list(accs)
            base = jnp.full((_LANES,), mi * _PU, jnp.int32)
            for u in range(_PU):
                idx = base + u
                mxs = plsc.load_gather(mx_v, [idx])
                mys = plsc.load_gather(my_v, [idx])
                mzs = plsc.load_gather(mz_v, [idx])
                aas = plsc.load_gather(aa_v, [idx])
                for i in range(_QV):
                    t = qx[i] * mxs
                    t = t + qy[i] * mys
                    t = t + qz[i] * mzs
                    s = aas + bb[i]
                    d = s - (t + t)
                    accs[i] = jnp.minimum(accs[i], d)
            return tuple(accs)

        init = tuple(jnp.full((_LANES,), jnp.inf, jnp.float32)
                     for _ in range(_QV))
        accs = lax.fori_loop(0, _MPAD // _PU, mbody, init, unroll=1)
        for i in range(_QV):
            total = total + accs[i]

    sum_v[...] = total
    pltpu.sync_copy(sum_v, out_hbm.at[pl.ds(w * _LANES, _LANES)])


def _chamfer_partials(meshf, pc):
    mesh_sc = plsc.VectorSubcoreMesh(core_axis_name="c", subcore_axis_name="s")
    kern = functools.partial(
        pl.kernel,
        mesh=mesh_sc,
        compiler_params=pltpu.CompilerParams(needs_layout_passes=False),
        out_type=jax.ShapeDtypeStruct((_NW * _LANES,), jnp.float32),
        scratch_types=[
            pltpu.VMEM((_MPAD,), jnp.float32),
            pltpu.VMEM((_MPAD,), jnp.float32),
            pltpu.VMEM((_MPAD,), jnp.float32),
            pltpu.VMEM((_MPAD,), jnp.float32),
            pltpu.VMEM((_QPW,), jnp.float32),
            pltpu.VMEM((_QPW,), jnp.float32),
            pltpu.VMEM((_QPW,), jnp.float32),
            pltpu.VMEM((_LANES,), jnp.float32),
        ],
    )(_chamfer_body)
    return kern(meshf, pc)


def kernel(network_mesh, pc):
    refined = _refine(network_mesh)
    meshf = refined.reshape(_B, _Z, _M)
    meshf = jnp.pad(meshf, ((0, 0), (0, 0), (0, _MPAD - _M)),
                    constant_values=_PADVAL)
    partials = _chamfer_partials(meshf.reshape(-1), pc.reshape(-1))
    return jnp.sum(partials) / jnp.float32(_B * _N)


# speedup vs baseline: 3.0894x; 3.0894x over previous
import functools

import jax
import jax.numpy as jnp
import numpy as np
from jax import lax
from jax.experimental import pallas as pl
from jax.experimental.pallas import tpu as pltpu
from jax.experimental.pallas import tpu_sc as plsc

_B = 2
_Z = 3
_G = 33
_F = 3
_R = (_G - 1) * _F + 1
_M = _R * _R
_N = 4096
_LANES = 16
_NC = 2
_NS = 16
_NW = _NC * _NS
_MPAD = 9472
_PADVAL = 1e8

_QV = 4
_PU = 8

_NSC = 1024
_NTC = _N - _NSC
_QPW = _NSC // _NS
_NB = 512
_MB = 256


def _interp_matrix() -> np.ndarray:
    w = np.zeros((_G, _R), dtype=np.float32)
    for p in range(_R):
        j = p // _F
        k = np.float32(p % _F) / np.float32(_F)
        jn = min(j + 1, _G - 1)
        w[j, p] += np.float32(1.0) - k
        w[jn, p] += k
    return w


def _refine_body(wt_ref, w_ref, m_ref, o_ref):
    m = m_ref[0]
    t = lax.dot_general(
        wt_ref[...], m, (((1,), (0,)), ((), ())),
        precision=lax.Precision.HIGHEST,
        preferred_element_type=jnp.float32)
    o_ref[0] = lax.dot_general(
        t, w_ref[...], (((1,), (0,)), ((), ())),
        precision=lax.Precision.HIGHEST,
        preferred_element_type=jnp.float32)


def _refine(network_mesh):
    mesh6 = network_mesh.reshape(_B * _Z, _G, _G)
    w = jnp.asarray(_interp_matrix())
    wt = w.T
    return pl.pallas_call(
        _refine_body,
        grid=(_B * _Z,),
        in_specs=[
            pl.BlockSpec((_R, _G), lambda i: (0, 0)),
            pl.BlockSpec((_G, _R), lambda i: (0, 0)),
            pl.BlockSpec((1, _G, _G), lambda i: (i, 0, 0)),
        ],
        out_specs=pl.BlockSpec((1, _R, _R), lambda i: (i, 0, 0)),
        out_shape=jax.ShapeDtypeStruct((_B * _Z, _R, _R), jnp.float32),
    )(wt, w, mesh6)


def _round_bf16(v):
    u = plsc.bitcast(v, jnp.uint32)
    lsb = jnp.bitwise_and(jnp.right_shift(u, jnp.uint32(16)), jnp.uint32(1))
    r = jnp.bitwise_and(u + jnp.uint32(0x7FFF) + lsb, jnp.uint32(0xFFFF0000))
    return plsc.bitcast(r, jnp.float32)


def _chamfer_body(meshf_hbm, pc_hbm, out_hbm,
                  mx_v, my_v, mz_v, aa_v, px_v, py_v, pz_v, sum_v):
    c = lax.axis_index("c")
    s = lax.axis_index("s")
    b = c
    w = s * _NC + c
    qoff = _NTC + s * _QPW

    pltpu.sync_copy(meshf_hbm.at[pl.ds((b * _Z + 0) * _MPAD, _MPAD)], mx_v)
    pltpu.sync_copy(meshf_hbm.at[pl.ds((b * _Z + 1) * _MPAD, _MPAD)], my_v)
    pltpu.sync_copy(meshf_hbm.at[pl.ds((b * _Z + 2) * _MPAD, _MPAD)], mz_v)
    pltpu.sync_copy(pc_hbm.at[pl.ds((b * _Z + 0) * _N + qoff, _QPW)], px_v)
    pltpu.sync_copy(pc_hbm.at[pl.ds((b * _Z + 1) * _N + qoff, _QPW)], py_v)
    pltpu.sync_copy(pc_hbm.at[pl.ds((b * _Z + 2) * _N + qoff, _QPW)], pz_v)

    def prep(vi, _):
        sl = pl.ds(vi * _LANES, _LANES)
        a = mx_v[sl]
        b2 = my_v[sl]
        cz = mz_v[sl]
        aa_v[sl] = a * a + b2 * b2 + cz * cz
        mx_v[sl] = _round_bf16(a)
        my_v[sl] = _round_bf16(b2)
        mz_v[sl] = _round_bf16(cz)
        return 0
    lax.fori_loop(0, _MPAD // _LANES, prep, 0, unroll=1)

    total = jnp.zeros((_LANES,), jnp.float32)
    n_groups = _QPW // (_QV * _LANES)
    for g in range(n_groups):
        qxr = [px_v[pl.ds((g * _QV + i) * _LANES, _LANES)] for i in range(_QV)]
        qyr = [py_v[pl.ds((g * _QV + i) * _LANES, _LANES)] for i in range(_QV)]
        qzr = [pz_v[pl.ds((g * _QV + i) * _LANES, _LANES)] for i in range(_QV)]
        bb = [qxr[i] * qxr[i] + qyr[i] * qyr[i] + qzr[i] * qzr[i]
              for i in range(_QV)]
        qx = [_round_bf16(qxr[i]) for i in range(_QV)]
        qy = [_round_bf16(qyr[i]) for i in range(_QV)]
        qz = [_round_bf16(qzr[i]) for i in range(_QV)]

        def mbody(mi, accs, qx=qx, qy=qy, qz=qz, bb=bb):
            accs = list(accs)
            base = jnp.full((_LANES,), mi * _PU, jnp.int32)
            for u in range(_PU):
                idx = base + u
                mxs = plsc.load_gather(mx_v, [idx])
                mys = plsc.load_gather(my_v, [idx])
                mzs = plsc.load_gather(mz_v, [idx])
                aas = plsc.load_gather(aa_v, [idx])
                for i in range(_QV):
                    t = qx[i] * mxs
                    t = t + qy[i] * mys
                    t = t + qz[i] * mzs
                    s = aas + bb[i]
                    d = s - (t + t)
                    accs[i] = jnp.minimum(accs[i], d)
            return tuple(accs)

        init = tuple(jnp.full((_LANES,), jnp.inf, jnp.float32)
                     for _ in range(_QV))
        accs = lax.fori_loop(0, _MPAD // _PU, mbody, init, unroll=1)
        for i in range(_QV):
            total = total + accs[i]

    sum_v[...] = total
    pltpu.sync_copy(sum_v, out_hbm.at[pl.ds(w * _LANES, _LANES)])


def _chamfer_partials(meshf, pc):
    mesh_sc = plsc.VectorSubcoreMesh(core_axis_name="c", subcore_axis_name="s")
    kern = functools.partial(
        pl.kernel,
        mesh=mesh_sc,
        compiler_params=pltpu.CompilerParams(needs_layout_passes=False),
        out_type=jax.ShapeDtypeStruct((_NW * _LANES,), jnp.float32),
        scratch_types=[
            pltpu.VMEM((_MPAD,), jnp.float32),
            pltpu.VMEM((_MPAD,), jnp.float32),
            pltpu.VMEM((_MPAD,), jnp.float32),
            pltpu.VMEM((_MPAD,), jnp.float32),
            pltpu.VMEM((_QPW,), jnp.float32),
            pltpu.VMEM((_QPW,), jnp.float32),
            pltpu.VMEM((_QPW,), jnp.float32),
            pltpu.VMEM((_LANES,), jnp.float32),
        ],
    )(_chamfer_body)
    return kern(meshf, pc)


def _tc_body(meshf_ref, meshbf_ref, pcs_ref, pcf_ref, o_ref, aa_s):
    nt = pl.program_id(1)

    @pl.when(nt == 0)
    def _():
        m = meshf_ref[0]
        aa_s[...] = jnp.sum(m * m, axis=0, keepdims=True)

    pcs = pcs_ref[0]
    runmin = jnp.full((_NB, _MB), jnp.inf, jnp.float32)
    for mt in range(_MPAD // _MB):
        msub = meshbf_ref[0, :, mt * _MB:(mt + 1) * _MB]
        t2 = lax.dot_general(pcs, msub, (((1,), (0,)), ((), ())),
                             preferred_element_type=jnp.float32)
        u = t2 + aa_s[0:1, mt * _MB:(mt + 1) * _MB]
        runmin = jnp.minimum(runmin, u)
    red = jnp.min(runmin, axis=1)
    pcf = pcf_ref[0]
    bb = jnp.sum(pcf * pcf, axis=0)
    o_ref[0, 0] = red + bb


def _tc_chamfer(meshf, mesh_bf, pcs_bf, pc_tc):
    return pl.pallas_call(
        _tc_body,
        grid=(_B, _NTC // _NB),
        in_specs=[
            pl.BlockSpec((1, _Z, _MPAD), lambda b, n: (b, 0, 0)),
            pl.BlockSpec((1, _Z, _MPAD), lambda b, n: (b, 0, 0)),
            pl.BlockSpec((1, _NB, _Z), lambda b, n: (b, n, 0)),
            pl.BlockSpec((1, _Z, _NB), lambda b, n: (b, 0, n)),
        ],
        out_specs=pl.BlockSpec(
            (1, 1, _NB), lambda b, n: (b * (_NTC // _NB) + n, 0, 0)),
        out_shape=jax.ShapeDtypeStruct((_B * (_NTC // _NB), 1, _NB),
                                       jnp.float32),
        scratch_shapes=[pltpu.VMEM((1, _MPAD), jnp.float32)],
    )(meshf, mesh_bf, pcs_bf, pc_tc)


def kernel(network_mesh, pc):
    refined = _refine(network_mesh)
    meshf = refined.reshape(_B, _Z, _M)
    meshf = jnp.pad(meshf, ((0, 0), (0, 0), (0, _MPAD - _M)),
                    constant_values=_PADVAL)
    mesh_bf = meshf.astype(jnp.bfloat16)
    pc_tc = pc[:, :, :_NTC]
    pcs_bf = jnp.transpose(-2.0 * pc_tc, (0, 2, 1)).astype(jnp.bfloat16)
    dist_tc = _tc_chamfer(meshf, mesh_bf, pcs_bf, pc_tc)
    partials = _chamfer_partials(meshf.reshape(-1), pc.reshape(-1))
    return (jnp.sum(dist_tc) + jnp.sum(partials)) / jnp.float32(_B * _N)


# speedup vs baseline: 8.7670x; 2.8377x over previous
import functools

import jax
import jax.numpy as jnp
import numpy as np
from jax import lax
from jax.experimental import pallas as pl
from jax.experimental.pallas import tpu as pltpu
from jax.experimental.pallas import tpu_sc as plsc

_B = 2
_Z = 3
_G = 33
_F = 3
_R = (_G - 1) * _F + 1
_M = _R * _R
_N = 4096
_LANES = 16
_NC = 2
_NS = 16
_NW = _NC * _NS
_MPAD = 9472
_PADVAL = 1e8

_QV = 4
_PU = 8

_NSC = 0
_NTC = _N - _NSC
_QPW = _NSC // _NS
_NB = 512
_MB = 256


def _interp_matrix() -> np.ndarray:
    w = np.zeros((_G, _R), dtype=np.float32)
    for p in range(_R):
        j = p // _F
        k = np.float32(p % _F) / np.float32(_F)
        jn = min(j + 1, _G - 1)
        w[j, p] += np.float32(1.0) - k
        w[jn, p] += k
    return w


def _refine_body(wt_ref, w_ref, m_ref, o_ref):
    m = m_ref[0]
    t = lax.dot_general(
        wt_ref[...], m, (((1,), (0,)), ((), ())),
        precision=lax.Precision.HIGHEST,
        preferred_element_type=jnp.float32)
    o_ref[0] = lax.dot_general(
        t, w_ref[...], (((1,), (0,)), ((), ())),
        precision=lax.Precision.HIGHEST,
        preferred_element_type=jnp.float32)


def _refine(network_mesh):
    mesh6 = network_mesh.reshape(_B * _Z, _G, _G)
    w = jnp.asarray(_interp_matrix())
    wt = w.T
    return pl.pallas_call(
        _refine_body,
        grid=(_B * _Z,),
        in_specs=[
            pl.BlockSpec((_R, _G), lambda i: (0, 0)),
            pl.BlockSpec((_G, _R), lambda i: (0, 0)),
            pl.BlockSpec((1, _G, _G), lambda i: (i, 0, 0)),
        ],
        out_specs=pl.BlockSpec((1, _R, _R), lambda i: (i, 0, 0)),
        out_shape=jax.ShapeDtypeStruct((_B * _Z, _R, _R), jnp.float32),
    )(wt, w, mesh6)


def _round_bf16(v):
    u = plsc.bitcast(v, jnp.uint32)
    lsb = jnp.bitwise_and(jnp.right_shift(u, jnp.uint32(16)), jnp.uint32(1))
    r = jnp.bitwise_and(u + jnp.uint32(0x7FFF) + lsb, jnp.uint32(0xFFFF0000))
    return plsc.bitcast(r, jnp.float32)


def _chamfer_body(meshf_hbm, pc_hbm, out_hbm,
                  mx_v, my_v, mz_v, aa_v, px_v, py_v, pz_v, sum_v):
    c = lax.axis_index("c")
    s = lax.axis_index("s")
    b = c
    w = s * _NC + c
    qoff = _NTC + s * _QPW

    pltpu.sync_copy(meshf_hbm.at[pl.ds((b * _Z + 0) * _MPAD, _MPAD)], mx_v)
    pltpu.sync_copy(meshf_hbm.at[pl.ds((b * _Z + 1) * _MPAD, _MPAD)], my_v)
    pltpu.sync_copy(meshf_hbm.at[pl.ds((b * _Z + 2) * _MPAD, _MPAD)], mz_v)
    pltpu.sync_copy(pc_hbm.at[pl.ds((b * _Z + 0) * _N + qoff, _QPW)], px_v)
    pltpu.sync_copy(pc_hbm.at[pl.ds((b * _Z + 1) * _N + qoff, _QPW)], py_v)
    pltpu.sync_copy(pc_hbm.at[pl.ds((b * _Z + 2) * _N + qoff, _QPW)], pz_v)

    def prep(vi, _):
        sl = pl.ds(vi * _LANES, _LANES)
        a = mx_v[sl]
        b2 = my_v[sl]
        cz = mz_v[sl]
        aa_v[sl] = a * a + b2 * b2 + cz * cz
        mx_v[sl] = _round_bf16(a)
        my_v[sl] = _round_bf16(b2)
        mz_v[sl] = _round_bf16(cz)
        return 0
    lax.fori_loop(0, _MPAD // _LANES, prep, 0, unroll=1)

    total = jnp.zeros((_LANES,), jnp.float32)
    n_groups = _QPW // (_QV * _LANES)
    for g in range(n_groups):
        qxr = [px_v[pl.ds((g * _QV + i) * _LANES, _LANES)] for i in range(_QV)]
        qyr = [py_v[pl.ds((g * _QV + i) * _LANES, _LANES)] for i in range(_QV)]
        qzr = [pz_v[pl.ds((g * _QV + i) * _LANES, _LANES)] for i in range(_QV)]
        bb = [qxr[i] * qxr[i] + qyr[i] * qyr[i] + qzr[i] * qzr[i]
              for i in range(_QV)]
        qx = [_round_bf16(qxr[i]) for i in range(_QV)]
        qy = [_round_bf16(qyr[i]) for i in range(_QV)]
        qz = [_round_bf16(qzr[i]) for i in range(_QV)]

        def mbody(mi, accs, qx=qx, qy=qy, qz=qz, bb=bb):
            accs = list(accs)
            base = jnp.full((_LANES,), mi * _PU, jnp.int32)
            for u in range(_PU):
                idx = base + u
                mxs = plsc.load_gather(mx_v, [idx])
                mys = plsc.load_gather(my_v, [idx])
                mzs = plsc.load_gather(mz_v, [idx])
                aas = plsc.load_gather(aa_v, [idx])
                for i in range(_QV):
                    t = qx[i] * mxs
                    t = t + qy[i] * mys
                    t = t + qz[i] * mzs
                    s = aas + bb[i]
                    d = s - (t + t)
                    accs[i] = jnp.minimum(accs[i], d)
            return tuple(accs)

        init = tuple(jnp.full((_LANES,), jnp.inf, jnp.float32)
                     for _ in range(_QV))
        accs = lax.fori_loop(0, _MPAD // _PU, mbody, init, unroll=1)
        for i in range(_QV):
            total = total + accs[i]

    sum_v[...] = total
    pltpu.sync_copy(sum_v, out_hbm.at[pl.ds(w * _LANES, _LANES)])


def _chamfer_partials(meshf, pc):
    mesh_sc = plsc.VectorSubcoreMesh(core_axis_name="c", subcore_axis_name="s")
    kern = functools.partial(
        pl.kernel,
        mesh=mesh_sc,
        compiler_params=pltpu.CompilerParams(needs_layout_passes=False),
        out_type=jax.ShapeDtypeStruct((_NW * _LANES,), jnp.float32),
        scratch_types=[
            pltpu.VMEM((_MPAD,), jnp.float32),
            pltpu.VMEM((_MPAD,), jnp.float32),
            pltpu.VMEM((_MPAD,), jnp.float32),
            pltpu.VMEM((_MPAD,), jnp.float32),
            pltpu.VMEM((_QPW,), jnp.float32),
            pltpu.VMEM((_QPW,), jnp.float32),
            pltpu.VMEM((_QPW,), jnp.float32),
            pltpu.VMEM((_LANES,), jnp.float32),
        ],
    )(_chamfer_body)
    return kern(meshf, pc)


def _tc_body(meshf_ref, meshbf_ref, pcs_ref, pcf_ref, o_ref, aa_s):
    nt = pl.program_id(1)

    @pl.when(nt == 0)
    def _():
        m = meshf_ref[0]
        aa_s[...] = jnp.sum(m * m, axis=0, keepdims=True)

    pcs = pcs_ref[0]
    runmin = jnp.full((_NB, _MB), jnp.inf, jnp.float32)
    for mt in range(_MPAD // _MB):
        msub = meshbf_ref[0, :, mt * _MB:(mt + 1) * _MB]
        t2 = lax.dot_general(pcs, msub, (((1,), (0,)), ((), ())),
                             preferred_element_type=jnp.float32)
        u = t2 + aa_s[0:1, mt * _MB:(mt + 1) * _MB]
        runmin = jnp.minimum(runmin, u)
    red = jnp.min(runmin, axis=1)
    pcf = pcf_ref[0]
    bb = jnp.sum(pcf * pcf, axis=0)
    o_ref[0, 0] = red + bb


def _tc_chamfer(meshf, mesh_bf, pcs_bf, pc_tc):
    return pl.pallas_call(
        _tc_body,
        grid=(_B, _NTC // _NB),
        in_specs=[
            pl.BlockSpec((1, _Z, _MPAD), lambda b, n: (b, 0, 0)),
            pl.BlockSpec((1, _Z, _MPAD), lambda b, n: (b, 0, 0)),
            pl.BlockSpec((1, _NB, _Z), lambda b, n: (b, n, 0)),
            pl.BlockSpec((1, _Z, _NB), lambda b, n: (b, 0, n)),
        ],
        out_specs=pl.BlockSpec(
            (1, 1, _NB), lambda b, n: (b * (_NTC // _NB) + n, 0, 0)),
        out_shape=jax.ShapeDtypeStruct((_B * (_NTC // _NB), 1, _NB),
                                       jnp.float32),
        scratch_shapes=[pltpu.VMEM((1, _MPAD), jnp.float32)],
    )(meshf, mesh_bf, pcs_bf, pc_tc)


def kernel(network_mesh, pc):
    refined = _refine(network_mesh)
    meshf = refined.reshape(_B, _Z, _M)
    meshf = jnp.pad(meshf, ((0, 0), (0, 0), (0, _MPAD - _M)),
                    constant_values=_PADVAL)
    mesh_bf = meshf.astype(jnp.bfloat16)
    pc_tc = pc[:, :, :_NTC]
    pcs_bf = jnp.transpose(-2.0 * pc_tc, (0, 2, 1)).astype(jnp.bfloat16)
    dist_tc = _tc_chamfer(meshf, mesh_bf, pcs_bf, pc_tc)
    total = jnp.sum(dist_tc)
    if _NSC:
        total = total + jnp.sum(
            _chamfer_partials(meshf.reshape(-1), pc.reshape(-1)))
    return total / jnp.float32(_B * _N)
